# baseline (speedup 1.0000x reference)
import jax
import jax.numpy as jnp
from jax import lax
from jax.experimental import pallas as pl
from jax.experimental.pallas import tpu as pltpu
from jax.experimental.pallas import tpu_sc as plsc

N = 10000
E = 320000
D = 128
HID = 64
K = 10

NC = 2
NS = 16
NW = NC * NS
EPW = E // NW
CH = 80
NCH = EPW // CH
NACC = N

STRIPE = 624
TAIL = N - NS * STRIPE

_MESH = plsc.VectorSubcoreMesh(core_axis_name="c", subcore_axis_name="s",
                               num_cores=NC, num_subcores=NS)


def _deg_body(dst3_hbm, out_hbm, idx2_v, ones_v, stage_v, acc_sh,
              semsc0, semsc1):
    c = lax.axis_index("c")
    s = lax.axis_index("s")
    w = c * NS + s
    rb = s * STRIPE
    for i in range(STRIPE // 16):
        stage_v[pl.ds(i * 16, 16)] = jnp.zeros((16,), jnp.float32)
    pltpu.sync_copy(stage_v, acc_sh.at[pl.ds(rb, STRIPE)])

    @pl.when(s == NS - 1)
    def _():
        pltpu.sync_copy(stage_v.at[pl.ds(0, TAIL)],
                        acc_sh.at[pl.ds(NS * STRIPE, TAIL)])

    for i in range(CH // 16):
        ones_v[pl.ds(i * 16, 16)] = jnp.ones((16,), jnp.float32)
    pltpu.sync_copy(dst3_hbm.at[w], idx2_v)
    plsc.subcore_barrier()

    def sca(i, sem):
        pltpu.async_copy(ones_v, acc_sh.at[idx2_v.at[i]], sem, add=True)

    def wsca(i, sem):
        pltpu.make_async_copy(ones_v, acc_sh.at[idx2_v.at[i]], sem).wait()

    sca(0, semsc0)
    sca(1, semsc1)

    def body(i, carry):
        j = 2 * i
        wsca(j, semsc0)
        sca(j + 2, semsc0)
        wsca(j + 1, semsc1)
        sca(j + 3, semsc1)
        return carry

    lax.fori_loop(0, (NCH - 3) // 2, body, 0)
    wsca(NCH - 3, semsc0)
    sca(NCH - 1, semsc0)
    wsca(NCH - 2, semsc1)
    wsca(NCH - 1, semsc0)
    plsc.subcore_barrier()
    pltpu.sync_copy(acc_sh.at[pl.ds(rb, STRIPE)], stage_v)
    pltpu.sync_copy(stage_v, out_hbm.at[pl.ds(c * N + rb, STRIPE)])

    @pl.when(s == NS - 1)
    def _():
        pltpu.sync_copy(acc_sh.at[pl.ds(NS * STRIPE, TAIL)],
                        stage_v.at[pl.ds(0, TAIL)])
        pltpu.sync_copy(stage_v.at[pl.ds(0, TAIL)],
                        out_hbm.at[pl.ds(c * N + NS * STRIPE, TAIL)])


_deg_call = pl.kernel(
    _deg_body,
    out_type=jax.ShapeDtypeStruct((NC * N,), jnp.float32),
    mesh=_MESH,
    scratch_types=[
        pltpu.VMEM((NCH, CH), jnp.int32),
        pltpu.VMEM((CH,), jnp.float32),
        pltpu.VMEM((STRIPE,), jnp.float32),
        pltpu.VMEM_SHARED((NACC,), jnp.float32),
        pltpu.SemaphoreType.DMA,
        pltpu.SemaphoreType.DMA,
    ],
)


OBUF = 208


def _spmm_body(p_hbm, src_hbm, dst_hbm, out_hbm,
               idxs0_v, idxs1_v, idxd0_v, idxd1_v, rows0_v, rows1_v, obuf_v,
               acc_sh, semis0, semis1, semid0, semid1, semg0, semg1):
    c = lax.axis_index("c")
    s = lax.axis_index("s")
    w = c * NS + s
    rb = s * STRIPE
    idxs_v = (idxs0_v, idxs1_v)
    idxd_v = (idxd0_v, idxd1_v)
    rows_v = (rows0_v, rows1_v)
    semis = (semis0, semis1)
    semid = (semid0, semid1)
    semg = (semg0, semg1)


    ebase = w * EPW

    def load_idx(j, b):
        pltpu.async_copy(src_hbm.at[pl.ds(ebase + j * CH, CH)], idxs_v[b],
                         semis[b])
        pltpu.async_copy(dst_hbm.at[pl.ds(ebase + j * CH, CH)], idxd_v[b],
                         semid[b])

    def wait_idx(j, b):
        pltpu.make_async_copy(src_hbm.at[pl.ds(ebase + j * CH, CH)],
                              idxs_v[b], semis[b]).wait()
        pltpu.make_async_copy(dst_hbm.at[pl.ds(ebase + j * CH, CH)],
                              idxd_v[b], semid[b]).wait()

    def gather(j, b):
        pltpu.async_copy(p_hbm.at[idxs_v[b]], rows_v[b], semg[b])

    def wait_gather(j, b):
        pltpu.make_async_copy(p_hbm.at[idxs_v[b]], rows_v[b], semg[b]).wait()

    def scatter(b):
        pltpu.sync_copy(rows_v[b], acc_sh.at[idxd_v[b]], add=True)

    load_idx(0, 0)
    load_idx(1, 1)

    def zrow(i, carry):
        for j in range(D // 16):
            obuf_v[i, pl.ds(j * 16, 16)] = jnp.zeros((16,), jnp.float32)
        return carry

    lax.fori_loop(0, OBUF, zrow, 0)
    for j in range(STRIPE // OBUF):
        pltpu.sync_copy(obuf_v, acc_sh.at[pl.ds(rb + j * OBUF, OBUF)])

    @pl.when(s == NS - 1)
    def _():
        pltpu.sync_copy(obuf_v.at[pl.ds(0, TAIL)],
                        acc_sh.at[pl.ds(NS * STRIPE, TAIL)])

    wait_idx(0, 0)
    gather(0, 0)
    plsc.subcore_barrier()

    def body(i, carry):
        j = 2 * i
        wait_idx(j + 1, 1)
        gather(j + 1, 1)
        wait_gather(j, 0)
        scatter(0)
        load_idx(j + 2, 0)
        wait_gather(j + 1, 1)
        scatter(1)
        wait_idx(j + 2, 0)
        gather(j + 2, 0)
        load_idx(j + 3, 1)
        return carry

    lax.fori_loop(0, (NCH - 3) // 2, body, 0)
    wait_idx(NCH - 2, 1)
    gather(NCH - 2, 1)
    wait_gather(NCH - 3, 0)
    scatter(0)
    load_idx(NCH - 1, 0)
    wait_gather(NCH - 2, 1)
    scatter(1)
    wait_idx(NCH - 1, 0)
    gather(NCH - 1, 0)
    wait_gather(NCH - 1, 0)
    scatter(0)
    plsc.subcore_barrier()
    for j in range(STRIPE // OBUF):
        pltpu.sync_copy(acc_sh.at[pl.ds(rb + j * OBUF, OBUF)], obuf_v)
        pltpu.sync_copy(obuf_v, out_hbm.at[c, pl.ds(rb + j * OBUF, OBUF)])

    @pl.when(s == NS - 1)
    def _():
        pltpu.sync_copy(acc_sh.at[pl.ds(NS * STRIPE, TAIL)],
                        obuf_v.at[pl.ds(0, TAIL)])
        pltpu.sync_copy(obuf_v.at[pl.ds(0, TAIL)],
                        out_hbm.at[c, pl.ds(NS * STRIPE, TAIL)])


_spmm_call = pl.kernel(
    _spmm_body,
    out_type=jax.ShapeDtypeStruct((NC, N, D), jnp.float32),
    mesh=_MESH,
    scratch_types=[
        pltpu.VMEM((CH,), jnp.int32),
        pltpu.VMEM((CH,), jnp.int32),
        pltpu.VMEM((CH,), jnp.int32),
        pltpu.VMEM((CH,), jnp.int32),
        pltpu.VMEM((CH, D), jnp.float32),
        pltpu.VMEM((CH, D), jnp.float32),
        pltpu.VMEM((OBUF, D), jnp.float32),
        pltpu.VMEM_SHARED((NACC, D), jnp.float32),
        pltpu.SemaphoreType.DMA,
        pltpu.SemaphoreType.DMA,
        pltpu.SemaphoreType.DMA,
        pltpu.SemaphoreType.DMA,
        pltpu.SemaphoreType.DMA,
        pltpu.SemaphoreType.DMA,
    ],
)


RB = 1000
GRID = N // RB


def _mlp_body(x_ref, w1t_ref, b1_ref, w2t_ref, b2_ref, h_ref):
    xb = x_ref[...]
    h1 = jnp.dot(xb, w1t_ref[...], preferred_element_type=jnp.float32)
    h1 = jnp.maximum(h1 + b1_ref[...], 0.0)
    h_ref[...] = (jnp.dot(h1, w2t_ref[...], preferred_element_type=jnp.float32)
                  + b2_ref[...])


def _mlp_call(x, w1t, b1r, w2t, b2r):
    return pl.pallas_call(
        _mlp_body,
        grid=(GRID,),
        in_specs=[
            pl.BlockSpec((RB, D), lambda i: (i, 0)),
            pl.BlockSpec((D, HID), lambda i: (0, 0)),
            pl.BlockSpec((1, HID), lambda i: (0, 0)),
            pl.BlockSpec((HID, D), lambda i: (0, 0)),
            pl.BlockSpec((1, D), lambda i: (0, 0)),
        ],
        out_specs=pl.BlockSpec((RB, D), lambda i: (i, 0)),
        out_shape=jax.ShapeDtypeStruct((N, D), jnp.float32),
    )(x, w1t, b1r, w2t, b2r)


def _scale_body(t_sm, h_ref, dis_ref, hid0_ref, p0_ref):
    h = h_ref[...]
    hid0_ref[...] = t_sm[0] * h
    p0_ref[...] = dis_ref[...] * h


def _scale_call(h, dis128, temp):
    blk = pl.BlockSpec((RB, D), lambda i: (i, 0))
    return pl.pallas_call(
        _scale_body,
        grid=(GRID,),
        in_specs=[pl.BlockSpec(memory_space=pltpu.SMEM), blk, blk],
        out_specs=[blk, blk],
        out_shape=[
            jax.ShapeDtypeStruct((N, D), jnp.float32),
            jax.ShapeDtypeStruct((N, D), jnp.float32),
        ],
    )(temp, h, dis128)


def _combine_body(tk_sm, acc_ref, p_ref, hid_ref, dis_ref, hidn_ref, pn_ref):
    t = acc_ref[0] + acc_ref[1] + p_ref[...]
    cur = dis_ref[...] * t
    hidn_ref[...] = hid_ref[...] + tk_sm[0] * cur
    pn_ref[...] = dis_ref[...] * cur


def _combine_call(acc2, p, hid, dis128, tk):
    blk = pl.BlockSpec((RB, D), lambda i: (i, 0))
    ablk = pl.BlockSpec((NC, RB, D), lambda i: (0, i, 0))
    return pl.pallas_call(
        _combine_body,
        grid=(GRID,),
        in_specs=[pl.BlockSpec(memory_space=pltpu.SMEM), ablk, blk, blk, blk],
        out_specs=[blk, blk],
        out_shape=[
            jax.ShapeDtypeStruct((N, D), jnp.float32),
            jax.ShapeDtypeStruct((N, D), jnp.float32),
        ],
    )(tk, acc2, p, hid, dis128)


def kernel(x, edge_index, W1, b1, W2, b2, temp):
    src = edge_index[0]
    dst = edge_index[1]

    deg2 = _deg_call(dst.reshape(NW, NCH, CH)).reshape(NC, N)
    h = _mlp_call(x, W1.T, b1.reshape(1, HID), W2.T, b2.reshape(1, D))
    deg = deg2[0] + deg2[1] + 1.0
    dis128 = jnp.broadcast_to(lax.rsqrt(deg)[:, None], (N, D))

    hidden, p = _scale_call(h, dis128, temp)
    for k in range(K):
        acc2 = _spmm_call(p, src, dst)
        hidden, p = _combine_call(acc2, p, hidden, dis128,
                                  temp[k + 1].reshape(1))
    return hidden


# speedup vs baseline: 1.0113x; 1.0113x over previous
import jax
import jax.numpy as jnp
from jax import lax
from jax.experimental import pallas as pl
from jax.experimental.pallas import tpu as pltpu
from jax.experimental.pallas import tpu_sc as plsc

N = 10000
E = 320000
D = 128
HID = 64
K = 10

NC = 2
NS = 16
NW = NC * NS
EPW = E // NW
CH = 80
NCH = EPW // CH
NACC = N

STRIPE = 624
TAIL = N - NS * STRIPE

_MESH = plsc.VectorSubcoreMesh(core_axis_name="c", subcore_axis_name="s",
                               num_cores=NC, num_subcores=NS)


def _deg_body(dst3_hbm, out_hbm, idx2_v, ones_v, stage_v, acc_sh,
              semsc0, semsc1):
    c = lax.axis_index("c")
    s = lax.axis_index("s")
    w = c * NS + s
    rb = s * STRIPE
    for i in range(STRIPE // 16):
        stage_v[pl.ds(i * 16, 16)] = jnp.zeros((16,), jnp.float32)
    pltpu.sync_copy(stage_v, acc_sh.at[pl.ds(rb, STRIPE)])

    @pl.when(s == NS - 1)
    def _():
        pltpu.sync_copy(stage_v.at[pl.ds(0, TAIL)],
                        acc_sh.at[pl.ds(NS * STRIPE, TAIL)])

    for i in range(CH // 16):
        ones_v[pl.ds(i * 16, 16)] = jnp.ones((16,), jnp.float32)
    pltpu.sync_copy(dst3_hbm.at[w], idx2_v)
    plsc.subcore_barrier()

    def sca(i, sem):
        pltpu.async_copy(ones_v, acc_sh.at[idx2_v.at[i]], sem, add=True)

    def wsca(i, sem):
        pltpu.make_async_copy(ones_v, acc_sh.at[idx2_v.at[i]], sem).wait()

    sca(0, semsc0)
    sca(1, semsc1)

    def body(i, carry):
        j = 2 * i
        wsca(j, semsc0)
        sca(j + 2, semsc0)
        wsca(j + 1, semsc1)
        sca(j + 3, semsc1)
        return carry

    lax.fori_loop(0, (NCH - 3) // 2, body, 0)
    wsca(NCH - 3, semsc0)
    sca(NCH - 1, semsc0)
    wsca(NCH - 2, semsc1)
    wsca(NCH - 1, semsc0)
    plsc.subcore_barrier()
    pltpu.sync_copy(acc_sh.at[pl.ds(rb, STRIPE)], stage_v)
    pltpu.sync_copy(stage_v, out_hbm.at[pl.ds(c * N + rb, STRIPE)])

    @pl.when(s == NS - 1)
    def _():
        pltpu.sync_copy(acc_sh.at[pl.ds(NS * STRIPE, TAIL)],
                        stage_v.at[pl.ds(0, TAIL)])
        pltpu.sync_copy(stage_v.at[pl.ds(0, TAIL)],
                        out_hbm.at[pl.ds(c * N + NS * STRIPE, TAIL)])


_deg_call = pl.kernel(
    _deg_body,
    out_type=jax.ShapeDtypeStruct((NC * N,), jnp.float32),
    mesh=_MESH,
    scratch_types=[
        pltpu.VMEM((NCH, CH), jnp.int32),
        pltpu.VMEM((CH,), jnp.float32),
        pltpu.VMEM((STRIPE,), jnp.float32),
        pltpu.VMEM_SHARED((NACC,), jnp.float32),
        pltpu.SemaphoreType.DMA,
        pltpu.SemaphoreType.DMA,
    ],
)


OBUF = 208


def _spmm_body(p_hbm, src_hbm, dst_hbm, out_hbm,
               idxs0_v, idxs1_v, idxd0_v, idxd1_v, rows0_v, rows1_v, obuf_v,
               acc_sh, semis0, semis1, semid0, semid1, semg0, semg1):
    c = lax.axis_index("c")
    s = lax.axis_index("s")
    w = c * NS + s
    rb = s * STRIPE
    idxs_v = (idxs0_v, idxs1_v)
    idxd_v = (idxd0_v, idxd1_v)
    rows_v = (rows0_v, rows1_v)
    semis = (semis0, semis1)
    semid = (semid0, semid1)
    semg = (semg0, semg1)


    ebase = w * EPW

    def load_idx(j, b):
        pltpu.async_copy(src_hbm.at[pl.ds(ebase + j * CH, CH)], idxs_v[b],
                         semis[b])
        pltpu.async_copy(dst_hbm.at[pl.ds(ebase + j * CH, CH)], idxd_v[b],
                         semid[b])

    def wait_idx(j, b):
        pltpu.make_async_copy(src_hbm.at[pl.ds(ebase + j * CH, CH)],
                              idxs_v[b], semis[b]).wait()
        pltpu.make_async_copy(dst_hbm.at[pl.ds(ebase + j * CH, CH)],
                              idxd_v[b], semid[b]).wait()

    def gather(j, b):
        pltpu.async_copy(p_hbm.at[idxs_v[b]], rows_v[b], semg[b])

    def wait_gather(j, b):
        pltpu.make_async_copy(p_hbm.at[idxs_v[b]], rows_v[b], semg[b]).wait()

    def scatter(b):
        pltpu.sync_copy(rows_v[b], acc_sh.at[idxd_v[b]], add=True)

    load_idx(0, 0)
    load_idx(1, 1)

    def zrow(i, carry):
        for j in range(D // 16):
            obuf_v[i, pl.ds(j * 16, 16)] = jnp.zeros((16,), jnp.float32)
        return carry

    lax.fori_loop(0, OBUF, zrow, 0)
    for j in range(STRIPE // OBUF):
        pltpu.sync_copy(obuf_v, acc_sh.at[pl.ds(rb + j * OBUF, OBUF)])

    @pl.when(s == NS - 1)
    def _():
        pltpu.sync_copy(obuf_v.at[pl.ds(0, TAIL)],
                        acc_sh.at[pl.ds(NS * STRIPE, TAIL)])

    wait_idx(0, 0)
    gather(0, 0)
    plsc.subcore_barrier()

    def body(i, carry):
        j = 2 * i
        wait_idx(j + 1, 1)
        gather(j + 1, 1)
        wait_gather(j, 0)
        scatter(0)
        load_idx(j + 2, 0)
        wait_gather(j + 1, 1)
        scatter(1)
        wait_idx(j + 2, 0)
        gather(j + 2, 0)
        load_idx(j + 3, 1)
        return carry

    lax.fori_loop(0, (NCH - 3) // 2, body, 0)
    wait_idx(NCH - 2, 1)
    gather(NCH - 2, 1)
    wait_gather(NCH - 3, 0)
    scatter(0)
    load_idx(NCH - 1, 0)
    wait_gather(NCH - 2, 1)
    scatter(1)
    wait_idx(NCH - 1, 0)
    gather(NCH - 1, 0)
    wait_gather(NCH - 1, 0)
    scatter(0)
    plsc.subcore_barrier()
    for j in range(STRIPE // OBUF):
        pltpu.sync_copy(acc_sh.at[pl.ds(rb + j * OBUF, OBUF)], obuf_v)
        pltpu.sync_copy(obuf_v, out_hbm.at[c, pl.ds(rb + j * OBUF, OBUF)])

    @pl.when(s == NS - 1)
    def _():
        pltpu.sync_copy(acc_sh.at[pl.ds(NS * STRIPE, TAIL)],
                        obuf_v.at[pl.ds(0, TAIL)])
        pltpu.sync_copy(obuf_v.at[pl.ds(0, TAIL)],
                        out_hbm.at[c, pl.ds(NS * STRIPE, TAIL)])


_spmm_call = pl.kernel(
    _spmm_body,
    out_type=jax.ShapeDtypeStruct((NC, N, D), jnp.float32),
    mesh=_MESH,
    scratch_types=[
        pltpu.VMEM((CH,), jnp.int32),
        pltpu.VMEM((CH,), jnp.int32),
        pltpu.VMEM((CH,), jnp.int32),
        pltpu.VMEM((CH,), jnp.int32),
        pltpu.VMEM((CH, D), jnp.float32),
        pltpu.VMEM((CH, D), jnp.float32),
        pltpu.VMEM((OBUF, D), jnp.float32),
        pltpu.VMEM_SHARED((NACC, D), jnp.float32),
        pltpu.SemaphoreType.DMA,
        pltpu.SemaphoreType.DMA,
        pltpu.SemaphoreType.DMA,
        pltpu.SemaphoreType.DMA,
        pltpu.SemaphoreType.DMA,
        pltpu.SemaphoreType.DMA,
    ],
)


RB = 2000
GRID = N // RB


def _mlp_body(x_ref, w1t_ref, b1_ref, w2t_ref, b2_ref, h_ref):
    xb = x_ref[...]
    h1 = jnp.dot(xb, w1t_ref[...], preferred_element_type=jnp.float32)
    h1 = jnp.maximum(h1 + b1_ref[...], 0.0)
    h_ref[...] = (jnp.dot(h1, w2t_ref[...], preferred_element_type=jnp.float32)
                  + b2_ref[...])


def _mlp_call(x, w1t, b1r, w2t, b2r):
    return pl.pallas_call(
        _mlp_body,
        grid=(GRID,),
        in_specs=[
            pl.BlockSpec((RB, D), lambda i: (i, 0)),
            pl.BlockSpec((D, HID), lambda i: (0, 0)),
            pl.BlockSpec((1, HID), lambda i: (0, 0)),
            pl.BlockSpec((HID, D), lambda i: (0, 0)),
            pl.BlockSpec((1, D), lambda i: (0, 0)),
        ],
        out_specs=pl.BlockSpec((RB, D), lambda i: (i, 0)),
        out_shape=jax.ShapeDtypeStruct((N, D), jnp.float32),
    )(x, w1t, b1r, w2t, b2r)


def _scale_body(t_sm, h_ref, dis_ref, hid0_ref, p0_ref):
    h = h_ref[...]
    hid0_ref[...] = t_sm[0] * h
    p0_ref[...] = dis_ref[...] * h


def _scale_call(h, dis128, temp):
    blk = pl.BlockSpec((RB, D), lambda i: (i, 0))
    return pl.pallas_call(
        _scale_body,
        grid=(GRID,),
        in_specs=[pl.BlockSpec(memory_space=pltpu.SMEM), blk, blk],
        out_specs=[blk, blk],
        out_shape=[
            jax.ShapeDtypeStruct((N, D), jnp.float32),
            jax.ShapeDtypeStruct((N, D), jnp.float32),
        ],
    )(temp, h, dis128)


def _combine_body(tk_sm, acc_ref, p_ref, hid_ref, dis_ref, hidn_ref, pn_ref):
    t = acc_ref[0] + acc_ref[1] + p_ref[...]
    cur = dis_ref[...] * t
    hidn_ref[...] = hid_ref[...] + tk_sm[0] * cur
    pn_ref[...] = dis_ref[...] * cur


def _combine_call(acc2, p, hid, dis128, tk):
    blk = pl.BlockSpec((RB, D), lambda i: (i, 0))
    ablk = pl.BlockSpec((NC, RB, D), lambda i: (0, i, 0))
    return pl.pallas_call(
        _combine_body,
        grid=(GRID,),
        in_specs=[pl.BlockSpec(memory_space=pltpu.SMEM), ablk, blk, blk, blk],
        out_specs=[blk, blk],
        out_shape=[
            jax.ShapeDtypeStruct((N, D), jnp.float32),
            jax.ShapeDtypeStruct((N, D), jnp.float32),
        ],
    )(tk, acc2, p, hid, dis128)


def kernel(x, edge_index, W1, b1, W2, b2, temp):
    src = edge_index[0]
    dst = edge_index[1]

    deg2 = _deg_call(dst.reshape(NW, NCH, CH)).reshape(NC, N)
    h = _mlp_call(x, W1.T, b1.reshape(1, HID), W2.T, b2.reshape(1, D))
    deg = deg2[0] + deg2[1] + 1.0
    dis128 = jnp.broadcast_to(lax.rsqrt(deg)[:, None], (N, D))

    hidden, p = _scale_call(h, dis128, temp)
    for k in range(K):
        acc2 = _spmm_call(p, src, dst)
        hidden, p = _combine_call(acc2, p, hidden, dis128,
                                  temp[k + 1].reshape(1))
    return hidden
